# baseline (speedup 1.0000x reference)
import jax
import jax.numpy as jnp
from jax.experimental import pallas as pl
import jax.experimental.pallas.tpu as pltpu

M_TILE = 512


def _gcn_kernel(x_ref, a_ref, w_ref, o_ref, wa_ref):
    @pl.when(pl.program_id(0) == 0)
    def _():
        wa = jnp.dot(
            w_ref[:], a_ref[:], preferred_element_type=jnp.float32
        )
        wa_ref[:] = wa.astype(jnp.bfloat16)

    r = jnp.dot(
        x_ref[:].astype(jnp.bfloat16),
        wa_ref[:],
        preferred_element_type=jnp.float32,
    )
    e = jnp.exp(r)
    o_ref[:] = e * (1.0 / jnp.sum(e, axis=-1, keepdims=True))


def kernel(inputs, normalized_adjacency, weights):
    n, din = inputs.shape
    dout = weights.shape[1]
    grid = (n // M_TILE,)
    return pl.pallas_call(
        _gcn_kernel,
        grid=grid,
        in_specs=[
            pl.BlockSpec((M_TILE, din), lambda i: (i, 0)),
            pl.BlockSpec((dout, n), lambda i: (0, 0)),
            pl.BlockSpec((din, dout), lambda i: (0, 0)),
        ],
        out_specs=pl.BlockSpec((M_TILE, n), lambda i: (i, 0)),
        out_shape=jax.ShapeDtypeStruct((n, normalized_adjacency.shape[0]), jnp.float32),
        scratch_shapes=[pltpu.VMEM((din, n), jnp.bfloat16)],
    )(inputs, normalized_adjacency, weights)


# speedup vs baseline: 1.0027x; 1.0027x over previous
import jax
import jax.numpy as jnp
from jax.experimental import pallas as pl
import jax.experimental.pallas.tpu as pltpu

M_TILE = 512


def _gcn_kernel(x_ref, a_ref, w_ref, o_ref, wa_ref):
    @pl.when(pl.program_id(0) == 0)
    def _():
        wa = jnp.dot(
            w_ref[:].astype(jnp.bfloat16),
            a_ref[:].astype(jnp.bfloat16),
            preferred_element_type=jnp.float32,
        )
        wa_ref[:] = wa.astype(jnp.bfloat16)

    r = jnp.dot(
        x_ref[:].astype(jnp.bfloat16),
        wa_ref[:],
        preferred_element_type=jnp.float32,
    )
    e = jnp.exp(r)
    o_ref[:] = e * (1.0 / jnp.sum(e, axis=-1, keepdims=True))


def kernel(inputs, normalized_adjacency, weights):
    n, din = inputs.shape
    dout = weights.shape[1]
    grid = (n // M_TILE,)
    return pl.pallas_call(
        _gcn_kernel,
        grid=grid,
        in_specs=[
            pl.BlockSpec((M_TILE, din), lambda i: (i, 0)),
            pl.BlockSpec((dout, n), lambda i: (0, 0)),
            pl.BlockSpec((din, dout), lambda i: (0, 0)),
        ],
        out_specs=pl.BlockSpec((M_TILE, n), lambda i: (i, 0)),
        out_shape=jax.ShapeDtypeStruct((n, normalized_adjacency.shape[0]), jnp.float32),
        scratch_shapes=[pltpu.VMEM((din, n), jnp.bfloat16)],
    )(inputs, normalized_adjacency, weights)
